# baseline (speedup 1.0000x reference)
import functools

import jax
import jax.numpy as jnp
import numpy as np
from jax import lax
from jax.experimental import pallas as pl
from jax.experimental.pallas import tpu as pltpu
from jax.experimental.pallas import tpu_sc as plsc

N_CTX = 7
PATTERN_BITS = N_CTX * 3
POS_BITS_IN = N_CTX * 5 + 3
POS_ADDR_BITS = 12
B = 262144

HP = 1 << 20
HQ = 1 << 11
PAD = 16384
TBL = HP + PAD
TQ = HQ + 256
WQ = TQ // 16

NPT = B // 16
CH = 8192


def _wiring():
    rng = np.random.default_rng(42)
    for _ in range(3):
        rng.permutation(PATTERN_BITS)
    pos_conn = np.stack(
        [rng.choice(POS_BITS_IN, size=POS_ADDR_BITS, replace=False) for _ in range(5)]
    )
    w_pos = np.zeros((POS_BITS_IN, 5), dtype=np.int32)
    for q in range(5):
        for k in range(POS_ADDR_BITS):
            w_pos[pos_conn[q, k], q] = 1 << (POS_ADDR_BITS - 1 - k)
    w_pat = (1 << np.arange(PATTERN_BITS, dtype=np.int32)).astype(np.int32)
    return w_pat, w_pos


_W_PAT, _W_POS = _wiring()


def _key_weights():
    w1 = np.zeros((16, PATTERN_BITS), dtype=np.float32)
    w2 = np.zeros((16, POS_BITS_IN), dtype=np.float32)
    for s in range(2):
        for c in range(3):
            w1[s * 8 + c, :] = _W_PAT
        for q in range(5):
            w2[s * 8 + 3 + q, :] = _W_POS[:, q]
    return w1, w2


_W1, _W2 = _key_weights()


_BB = 8192


def _addr_body(tbits_ref, pbits_ref, w1_ref, w2_ref, sidx_ref, gidx_ref):
    tb = tbits_ref[...].astype(jnp.float32)
    pb = pbits_ref[...].astype(jnp.float32)
    k16 = (jnp.dot(w1_ref[...], tb, preferred_element_type=jnp.float32)
           + jnp.dot(w2_ref[...], pb, preferred_element_type=jnp.float32))
    ki = k16.astype(jnp.int32)
    row = lax.broadcasted_iota(jnp.int32, ki.shape, 0)
    is_pat = (row & 7) < 3
    h = jnp.where(is_pat, HP, HQ)
    lo = jnp.where(row >= 8, h, 0)
    inw = (ki >= lo) & (ki < lo + h)
    rel = ki - lo
    half = jnp.where(is_pat, 8192, 128)
    spread = jnp.bitwise_and(ki, half - 1)
    sidx_ref[...] = jnp.where(inw, rel, h + spread)
    gidx_ref[...] = jnp.where(inw, rel, h + half + spread)


_addr_call = pl.pallas_call(
    _addr_body,
    grid=(B // _BB,),
    in_specs=[
        pl.BlockSpec((PATTERN_BITS, _BB), lambda i: (0, i)),
        pl.BlockSpec((POS_BITS_IN, _BB), lambda i: (0, i)),
        pl.BlockSpec((16, PATTERN_BITS), lambda i: (0, 0)),
        pl.BlockSpec((16, POS_BITS_IN), lambda i: (0, 0)),
    ],
    out_specs=[
        pl.BlockSpec((16, _BB), lambda i: (0, i)),
        pl.BlockSpec((16, _BB), lambda i: (0, i)),
    ],
    out_shape=[
        jax.ShapeDtypeStruct((16, B), jnp.int32),
        jax.ShapeDtypeStruct((16, B), jnp.int32),
    ],
)


def _sc_scatter_body(sidx_hbm, gidx_hbm, vals_hbm, zeros_hbm, out_hbm,
                     table, ibuf, gbuf, vbuf, pbuf, zbuf, qtab, qshared,
                     redbuf, sem):
    sc = lax.axis_index("c")
    tile = lax.axis_index("s")
    b0 = tile * NPT

    def scatter_add(row, c):
        for ck in range(NPT // CH):
            pltpu.sync_copy(sidx_hbm.at[row, pl.ds(b0 + ck * CH, CH)], ibuf)
            pltpu.sync_copy(vals_hbm.at[c, pl.ds(b0 + ck * CH, CH)], vbuf)
            pltpu.sync_copy(vbuf, table.at[ibuf], add=True)

    def gather_out(row):
        for ck in range(NPT // CH):
            pltpu.sync_copy(gidx_hbm.at[row, pl.ds(b0 + ck * CH, CH)], gbuf)
            pltpu.async_copy(table.at[gbuf], pbuf, sem).wait()
            pltpu.sync_copy(pbuf, out_hbm.at[row, pl.ds(b0 + ck * CH, CH)])

    pltpu.sync_copy(zeros_hbm, zbuf)
    zn = (HP + PAD) // 16
    for j in range(0, zn, CH):
        w = min(CH, zn - j)
        pltpu.sync_copy(zbuf.at[pl.ds(0, w)], table.at[pl.ds(tile * zn + j, w)])
    plsc.subcore_barrier()
    for c in range(3):
        row = sc * 8 + c
        scatter_add(row, c)
        plsc.subcore_barrier()
        gather_out(row)
        plsc.subcore_barrier()
        if c < 2:
            for ck in range(NPT // CH):
                pltpu.sync_copy(sidx_hbm.at[row, pl.ds(b0 + ck * CH, CH)], ibuf)
                pltpu.sync_copy(zbuf, table.at[ibuf])
            plsc.subcore_barrier()
    zero16 = jnp.zeros((16,), jnp.float32)
    for c in range(3, 8):
        row = sc * 8 + c

        def zr(i, _):
            qtab[pl.ds(i * 16, 16)] = zero16
            return 0
        lax.fori_loop(0, TQ // 16, zr, 0)
        for ck in range(NPT // CH):
            pltpu.sync_copy(sidx_hbm.at[row, pl.ds(b0 + ck * CH, CH)], ibuf)
            pltpu.sync_copy(vals_hbm.at[c, pl.ds(b0 + ck * CH, CH)], vbuf)

            def sca(i, _):
                idx = ibuf[pl.ds(i * 16, 16)]
                v = vbuf[pl.ds(i * 16, 16)]
                plsc.addupdate_scatter(qtab, [idx], v)
                return 0
            lax.fori_loop(0, CH // 16, sca, 0)
        pltpu.sync_copy(qtab, qshared.at[pl.ds(tile * TQ, TQ)])
        plsc.subcore_barrier()
        for r in range(16):
            pltpu.sync_copy(qshared.at[pl.ds(r * TQ + tile * WQ, WQ)],
                            redbuf.at[pl.ds(r * WQ, WQ)])

        def red(j, _):
            acc = redbuf[pl.ds(j * 16, 16)]
            for r in range(1, 16):
                acc = acc + redbuf[pl.ds(r * WQ + j * 16, 16)]
            redbuf[pl.ds(j * 16, 16)] = acc
            return 0
        lax.fori_loop(0, WQ // 16, red, 0)
        pltpu.sync_copy(redbuf.at[pl.ds(0, WQ)],
                        qshared.at[pl.ds(16 * TQ + tile * WQ, WQ)])
        plsc.subcore_barrier()
        pltpu.sync_copy(qshared.at[pl.ds(16 * TQ, TQ)], qtab)
        plsc.subcore_barrier()
        for ck in range(NPT // CH):
            pltpu.sync_copy(gidx_hbm.at[row, pl.ds(b0 + ck * CH, CH)], gbuf)

            def gat(i, _):
                idx = gbuf[pl.ds(i * 16, 16)]
                pbuf[pl.ds(i * 16, 16)] = plsc.load_gather(qtab, [idx])
                return 0
            lax.fori_loop(0, CH // 16, gat, 0)
            pltpu.sync_copy(pbuf, out_hbm.at[row, pl.ds(b0 + ck * CH, CH)])


_sc_call_cache = []


def _sc_call():
    if not _sc_call_cache:
        _sc_call_cache.append(pl.kernel(
            _sc_scatter_body,
            out_type=jax.ShapeDtypeStruct((16, B), jnp.float32),
            mesh=plsc.VectorSubcoreMesh(core_axis_name="c", subcore_axis_name="s"),
            compiler_params=pltpu.CompilerParams(needs_layout_passes=False),
            scratch_types=[
                pltpu.VMEM_SHARED((TBL,), jnp.float32),
                pltpu.VMEM((CH,), jnp.int32),
                pltpu.VMEM((CH,), jnp.int32),
                pltpu.VMEM((CH,), jnp.float32),
                pltpu.VMEM((CH,), jnp.float32),
                pltpu.VMEM((CH,), jnp.float32),
                pltpu.VMEM((TQ,), jnp.float32),
                pltpu.VMEM_SHARED((17 * TQ,), jnp.float32),
                pltpu.VMEM((16 * WQ,), jnp.float32),
                pltpu.SemaphoreType.DMA,
            ],
        ))
    return _sc_call_cache[0]


_BM = 8192


def _merge_body(in_ref, out_ref):
    x = in_ref[...]
    out_ref[...] = x[0:8, :] + x[8:16, :]


_merge_call = pl.pallas_call(
    _merge_body,
    grid=(B // _BM,),
    in_specs=[pl.BlockSpec((16, _BM), lambda i: (0, i))],
    out_specs=pl.BlockSpec((8, _BM), lambda i: (0, i)),
    out_shape=jax.ShapeDtypeStruct((8, B), jnp.float32),
)


def kernel(x_type_bits, x_pos_bits, target_type, target_pos, pattern_mem, position_mem):
    sidx, gidx = _addr_call(
        x_type_bits.T, x_pos_bits.T, jnp.asarray(_W1), jnp.asarray(_W2))
    vals = jnp.concatenate([target_type.T, target_pos.T], axis=0)
    zeros = jnp.zeros((CH,), jnp.float32)
    partials = _sc_call()(sidx, gidx, vals, zeros)
    return _merge_call(partials).T


# speedup vs baseline: 1.2096x; 1.2096x over previous
import functools

import jax
import jax.numpy as jnp
import numpy as np
from jax import lax
from jax.experimental import pallas as pl
from jax.experimental.pallas import tpu as pltpu
from jax.experimental.pallas import tpu_sc as plsc

N_CTX = 7
PATTERN_BITS = N_CTX * 3
POS_BITS_IN = N_CTX * 5 + 3
POS_ADDR_BITS = 12
B = 262144

HP = 1 << 20
HQ = 1 << 11
PAD = 16384
TBL = HP + PAD

NPT = B // 16
CH = 16384


def _wiring():
    rng = np.random.default_rng(42)
    for _ in range(3):
        rng.permutation(PATTERN_BITS)
    pos_conn = np.stack(
        [rng.choice(POS_BITS_IN, size=POS_ADDR_BITS, replace=False) for _ in range(5)]
    )
    w_pos = np.zeros((POS_BITS_IN, 5), dtype=np.int32)
    for q in range(5):
        for k in range(POS_ADDR_BITS):
            w_pos[pos_conn[q, k], q] = 1 << (POS_ADDR_BITS - 1 - k)
    w_pat = (1 << np.arange(PATTERN_BITS, dtype=np.int32)).astype(np.int32)
    return w_pat, w_pos


_W_PAT, _W_POS = _wiring()


def _key_weights():
    w1 = np.zeros((16, PATTERN_BITS), dtype=np.float32)
    w2 = np.zeros((16, POS_BITS_IN), dtype=np.float32)
    for s in range(2):
        for c in range(3):
            w1[s * 8 + c, :] = _W_PAT
        for q in range(5):
            w2[s * 8 + 3 + q, :] = _W_POS[:, q]
    return w1, w2


_W1, _W2 = _key_weights()


_BB = 8192


def _addr_body(tbits_ref, pbits_ref, w1_ref, w2_ref, sidx_ref, gidx_ref):
    tb = tbits_ref[...].astype(jnp.float32)
    pb = pbits_ref[...].astype(jnp.float32)
    k16 = (jnp.dot(w1_ref[...], tb, preferred_element_type=jnp.float32)
           + jnp.dot(w2_ref[...], pb, preferred_element_type=jnp.float32))
    ki = k16.astype(jnp.int32)
    row = lax.broadcasted_iota(jnp.int32, ki.shape, 0)
    h = jnp.where((row & 7) < 3, HP, HQ)
    lo = jnp.where(row >= 8, h, 0)
    inw = (ki >= lo) & (ki < lo + h)
    rel = ki - lo
    spread = jnp.bitwise_and(ki, 8191)
    sidx_ref[...] = jnp.where(inw, rel, h + spread)
    gidx_ref[...] = jnp.where(inw, rel, h + 8192 + spread)


_addr_call = pl.pallas_call(
    _addr_body,
    grid=(B // _BB,),
    in_specs=[
        pl.BlockSpec((PATTERN_BITS, _BB), lambda i: (0, i)),
        pl.BlockSpec((POS_BITS_IN, _BB), lambda i: (0, i)),
        pl.BlockSpec((16, PATTERN_BITS), lambda i: (0, 0)),
        pl.BlockSpec((16, POS_BITS_IN), lambda i: (0, 0)),
    ],
    out_specs=[
        pl.BlockSpec((16, _BB), lambda i: (0, i)),
        pl.BlockSpec((16, _BB), lambda i: (0, i)),
    ],
    out_shape=[
        jax.ShapeDtypeStruct((16, B), jnp.int32),
        jax.ShapeDtypeStruct((16, B), jnp.int32),
    ],
)


def _sc_scatter_body(sidx_hbm, gidx_hbm, vals_hbm, zeros_hbm, out_hbm,
                     table, ibuf, vbuf, zbuf, sem):
    sc = lax.axis_index("c")
    tile = lax.axis_index("s")
    b0 = tile * NPT

    def scatter_add(row, c):
        pltpu.sync_copy(sidx_hbm.at[row, pl.ds(b0, NPT)], ibuf)
        pltpu.sync_copy(vals_hbm.at[c, pl.ds(b0, NPT)], vbuf)
        pltpu.sync_copy(vbuf, table.at[ibuf], add=True)

    def gather_out(row):
        pltpu.sync_copy(gidx_hbm.at[row, pl.ds(b0, NPT)], ibuf)
        pltpu.async_copy(table.at[ibuf], vbuf, sem).wait()
        pltpu.sync_copy(vbuf, out_hbm.at[row, pl.ds(b0, NPT)])

    pltpu.sync_copy(zeros_hbm, zbuf)
    zn = (HP + PAD) // 16
    for j in range(0, zn, CH):
        w = min(CH, zn - j)
        pltpu.sync_copy(zbuf.at[pl.ds(0, w)], table.at[pl.ds(tile * zn + j, w)])
    plsc.subcore_barrier()
    for c in range(3):
        row = sc * 8 + c
        scatter_add(row, c)
        plsc.subcore_barrier()
        gather_out(row)
        plsc.subcore_barrier()
        if c < 2:
            pltpu.sync_copy(sidx_hbm.at[row, pl.ds(b0, NPT)], ibuf)
            pltpu.sync_copy(zbuf, table.at[ibuf])
            plsc.subcore_barrier()
    zq = (HQ + PAD) // 16
    for c in range(3, 8):
        row = sc * 8 + c
        pltpu.sync_copy(zbuf.at[pl.ds(0, zq)], table.at[pl.ds(tile * zq, zq)])
        plsc.subcore_barrier()
        scatter_add(row, c)
        plsc.subcore_barrier()
        gather_out(row)
        plsc.subcore_barrier()


_sc_call_cache = []


def _sc_call():
    if not _sc_call_cache:
        _sc_call_cache.append(pl.kernel(
            _sc_scatter_body,
            out_type=jax.ShapeDtypeStruct((16, B), jnp.float32),
            mesh=plsc.VectorSubcoreMesh(core_axis_name="c", subcore_axis_name="s"),
            scratch_types=[
                pltpu.VMEM_SHARED((TBL,), jnp.float32),
                pltpu.VMEM((CH,), jnp.int32),
                pltpu.VMEM((CH,), jnp.float32),
                pltpu.VMEM((CH,), jnp.float32),
                pltpu.SemaphoreType.DMA,
            ],
        ))
    return _sc_call_cache[0]


_BM = 8192


def _merge_body(in_ref, out_ref):
    x = in_ref[...]
    out_ref[...] = x[0:8, :] + x[8:16, :]


_merge_call = pl.pallas_call(
    _merge_body,
    grid=(B // _BM,),
    in_specs=[pl.BlockSpec((16, _BM), lambda i: (0, i))],
    out_specs=pl.BlockSpec((8, _BM), lambda i: (0, i)),
    out_shape=jax.ShapeDtypeStruct((8, B), jnp.float32),
)


def kernel(x_type_bits, x_pos_bits, target_type, target_pos, pattern_mem, position_mem):
    sidx, gidx = _addr_call(
        x_type_bits.T, x_pos_bits.T, jnp.asarray(_W1), jnp.asarray(_W2))
    vals = jnp.concatenate([target_type.T, target_pos.T], axis=0)
    zeros = jnp.zeros((CH,), jnp.float32)
    partials = _sc_call()(sidx, gidx, vals, zeros)
    return _merge_call(partials).T


# speedup vs baseline: 1.2134x; 1.0032x over previous
import functools

import jax
import jax.numpy as jnp
import numpy as np
from jax import lax
from jax.experimental import pallas as pl
from jax.experimental.pallas import tpu as pltpu
from jax.experimental.pallas import tpu_sc as plsc

N_CTX = 7
PATTERN_BITS = N_CTX * 3
POS_BITS_IN = N_CTX * 5 + 3
POS_ADDR_BITS = 12
B = 262144

HP = 1 << 20
HQ = 1 << 11
PAD = 16384
QR = HQ + 256
QB0 = HP + PAD
TBL = QB0 + 5 * QR

NPT = B // 16
CH = 16384


def _wiring():
    rng = np.random.default_rng(42)
    for _ in range(3):
        rng.permutation(PATTERN_BITS)
    pos_conn = np.stack(
        [rng.choice(POS_BITS_IN, size=POS_ADDR_BITS, replace=False) for _ in range(5)]
    )
    w_pos = np.zeros((POS_BITS_IN, 5), dtype=np.int32)
    for q in range(5):
        for k in range(POS_ADDR_BITS):
            w_pos[pos_conn[q, k], q] = 1 << (POS_ADDR_BITS - 1 - k)
    w_pat = (1 << np.arange(PATTERN_BITS, dtype=np.int32)).astype(np.int32)
    return w_pat, w_pos


_W_PAT, _W_POS = _wiring()


def _key_weights():
    w1 = np.zeros((16, PATTERN_BITS), dtype=np.float32)
    w2 = np.zeros((16, POS_BITS_IN), dtype=np.float32)
    for s in range(2):
        for c in range(3):
            w1[s * 8 + c, :] = _W_PAT
        for q in range(5):
            w2[s * 8 + 3 + q, :] = _W_POS[:, q]
    return w1, w2


_W1, _W2 = _key_weights()


_BB = 8192


def _addr_body(tbits_ref, pbits_ref, w1_ref, w2_ref, sidx_ref, gidx_ref):
    tb = tbits_ref[...].astype(jnp.float32)
    pb = pbits_ref[...].astype(jnp.float32)
    k16 = (jnp.dot(w1_ref[...], tb, preferred_element_type=jnp.float32)
           + jnp.dot(w2_ref[...], pb, preferred_element_type=jnp.float32))
    ki = k16.astype(jnp.int32)
    row = lax.broadcasted_iota(jnp.int32, ki.shape, 0)
    is_pat = (row & 7) < 3
    h = jnp.where(is_pat, HP, HQ)
    lo = jnp.where(row >= 8, h, 0)
    inw = (ki >= lo) & (ki < lo + h)
    rel = ki - lo
    half = jnp.where(is_pat, 8192, 128)
    spread = jnp.bitwise_and(ki, half - 1)
    base = jnp.where(is_pat, 0, QB0 + ((row & 7) - 3) * QR)
    sidx_ref[...] = base + jnp.where(inw, rel, h + spread)
    gidx_ref[...] = base + jnp.where(inw, rel, h + half + spread)


_addr_call = pl.pallas_call(
    _addr_body,
    grid=(B // _BB,),
    in_specs=[
        pl.BlockSpec((PATTERN_BITS, _BB), lambda i: (0, i)),
        pl.BlockSpec((POS_BITS_IN, _BB), lambda i: (0, i)),
        pl.BlockSpec((16, PATTERN_BITS), lambda i: (0, 0)),
        pl.BlockSpec((16, POS_BITS_IN), lambda i: (0, 0)),
    ],
    out_specs=[
        pl.BlockSpec((16, _BB), lambda i: (0, i)),
        pl.BlockSpec((16, _BB), lambda i: (0, i)),
    ],
    out_shape=[
        jax.ShapeDtypeStruct((16, B), jnp.int32),
        jax.ShapeDtypeStruct((16, B), jnp.int32),
    ],
)


def _sc_scatter_body(sidx_hbm, gidx_hbm, vals_hbm, zeros_hbm, out_hbm,
                     table, ibuf, vbuf, zbuf, sem):
    sc = lax.axis_index("c")
    tile = lax.axis_index("s")
    b0 = tile * NPT

    def scatter_add(row, c):
        pltpu.sync_copy(sidx_hbm.at[row, pl.ds(b0, NPT)], ibuf)
        pltpu.sync_copy(vals_hbm.at[c, pl.ds(b0, NPT)], vbuf)
        pltpu.sync_copy(vbuf, table.at[ibuf], add=True)

    def gather_out(row):
        pltpu.sync_copy(gidx_hbm.at[row, pl.ds(b0, NPT)], ibuf)
        pltpu.async_copy(table.at[ibuf], vbuf, sem).wait()
        pltpu.sync_copy(vbuf, out_hbm.at[row, pl.ds(b0, NPT)])

    pltpu.sync_copy(zeros_hbm, zbuf)
    zn = TBL // 16
    for j in range(0, zn, CH):
        w = min(CH, zn - j)
        pltpu.sync_copy(zbuf.at[pl.ds(0, w)], table.at[pl.ds(tile * zn + j, w)])
    plsc.subcore_barrier()
    for c in range(3):
        row = sc * 8 + c
        scatter_add(row, c)
        plsc.subcore_barrier()
        gather_out(row)
        plsc.subcore_barrier()
        if c < 2:
            pltpu.sync_copy(sidx_hbm.at[row, pl.ds(b0, NPT)], ibuf)
            pltpu.sync_copy(zbuf, table.at[ibuf])
            plsc.subcore_barrier()
    for c in range(3, 8):
        scatter_add(sc * 8 + c, c)
    plsc.subcore_barrier()
    for c in range(3, 8):
        gather_out(sc * 8 + c)


_sc_call_cache = []


def _sc_call():
    if not _sc_call_cache:
        _sc_call_cache.append(pl.kernel(
            _sc_scatter_body,
            out_type=jax.ShapeDtypeStruct((16, B), jnp.float32),
            mesh=plsc.VectorSubcoreMesh(core_axis_name="c", subcore_axis_name="s"),
            scratch_types=[
                pltpu.VMEM_SHARED((TBL,), jnp.float32),
                pltpu.VMEM((CH,), jnp.int32),
                pltpu.VMEM((CH,), jnp.float32),
                pltpu.VMEM((CH,), jnp.float32),
                pltpu.SemaphoreType.DMA,
            ],
        ))
    return _sc_call_cache[0]


_BM = 8192


def _merge_body(in_ref, out_ref):
    x = in_ref[...]
    out_ref[...] = x[0:8, :] + x[8:16, :]


_merge_call = pl.pallas_call(
    _merge_body,
    grid=(B // _BM,),
    in_specs=[pl.BlockSpec((16, _BM), lambda i: (0, i))],
    out_specs=pl.BlockSpec((8, _BM), lambda i: (0, i)),
    out_shape=jax.ShapeDtypeStruct((8, B), jnp.float32),
)


def kernel(x_type_bits, x_pos_bits, target_type, target_pos, pattern_mem, position_mem):
    sidx, gidx = _addr_call(
        x_type_bits.T, x_pos_bits.T, jnp.asarray(_W1), jnp.asarray(_W2))
    vals = jnp.concatenate([target_type.T, target_pos.T], axis=0)
    zeros = jnp.zeros((CH,), jnp.float32)
    partials = _sc_call()(sidx, gidx, vals, zeros)
    return _merge_call(partials).T
